# baseline (speedup 1.0000x reference)
import functools

import jax
import jax.numpy as jnp
from jax import lax
from jax.experimental import pallas as pl
from jax.experimental.pallas import tpu as pltpu
from jax.experimental.pallas import tpu_sc as plsc

B = 16384
D = 128
NC = 2
NS = 16
NW = NC * NS
ROWS_PER_W = B // NW
CHUNK = 128
NCHUNK = ROWS_PER_W // CHUNK
GROUPS = CHUNK // 16


def _body(gu_hbm, gi_hbm, out_hbm, gu_v0, gu_v1, gi_v0, gi_v1, out_v,
          sem_u, sem_i):
    wid = lax.axis_index("s") * NC + lax.axis_index("c")
    base = wid * ROWS_PER_W
    gu_bufs = (gu_v0, gu_v1)
    gi_bufs = (gi_v0, gi_v1)

    def start(c, buf):
        r0 = base + c * CHUNK
        cu = pltpu.make_async_copy(
            gu_hbm.at[pl.ds(r0, CHUNK)], gu_bufs[buf], sem_u)
        ci = pltpu.make_async_copy(
            gi_hbm.at[pl.ds(r0, CHUNK)], gi_bufs[buf], sem_i)
        cu.start()
        ci.start()
        return cu, ci

    pending = start(0, 0)
    for c in range(NCHUNK):
        buf = c % 2
        cu, ci = pending
        if c + 1 < NCHUNK:
            pending = start(c + 1, 1 - buf)
        cu.wait()
        ci.wait()
        gu_b = gu_bufs[buf]
        gi_b = gi_bufs[buf]

        lane = lax.iota(jnp.int32, 16)

        def group_body(g, _):
            vs = []
            for rr in range(16):
                r = g * 16 + rr
                acc = gu_b[r, pl.ds(0, 16)] * gi_b[r, pl.ds(0, 16)]
                for j in range(1, D // 16):
                    acc = acc + (gu_b[r, pl.ds(j * 16, 16)]
                                 * gi_b[r, pl.ds(j * 16, 16)])
                vs.append(acc)
            for k in range(4):
                bit = 1 << k
                mask = (lane & bit) == 0
                perm = lane ^ bit
                vs = [jnp.where(mask, x, y) + jnp.where(mask, y, x)[perm]
                      for x, y in zip(vs[0::2], vs[1::2])]
            out_v[pl.ds(c * CHUNK + g * 16, 16)] = vs[0]
            return 0

        lax.fori_loop(0, GROUPS, group_body, 0)

    pltpu.sync_copy(out_v, out_hbm.at[pl.ds(base, ROWS_PER_W)])


@jax.jit
def kernel(gu, gi):
    mesh = plsc.VectorSubcoreMesh(core_axis_name="c", subcore_axis_name="s")
    f = functools.partial(
        pl.kernel,
        out_type=jax.ShapeDtypeStruct((B,), jnp.float32),
        mesh=mesh,
        scratch_types=[
            pltpu.VMEM((CHUNK, D), jnp.float32),
            pltpu.VMEM((CHUNK, D), jnp.float32),
            pltpu.VMEM((CHUNK, D), jnp.float32),
            pltpu.VMEM((CHUNK, D), jnp.float32),
            pltpu.VMEM((ROWS_PER_W,), jnp.float32),
            pltpu.SemaphoreType.DMA,
            pltpu.SemaphoreType.DMA,
        ],
    )(_body)
    return f(gu, gi)
